# baseline (speedup 1.0000x reference)
import functools

import jax
import jax.numpy as jnp
from jax import lax
from jax.experimental import pallas as pl
from jax.experimental.pallas import tpu as pltpu
from jax.experimental.pallas import tpu_sc as plsc

NW = 32
NG = 4
NT = 2


def _pos_encoding(length, d_model):
    depth = d_model / 2
    pos = jnp.arange(0, length, dtype=jnp.float32)[:, None]
    i = jnp.arange(0, depth, dtype=jnp.float32)
    angle = pos / jnp.power(10000.0, 2.0 * i / depth)
    return jnp.concatenate([jnp.sin(angle), jnp.cos(angle)], axis=-1)


def kernel(x, W):
    B, L = x.shape
    V, D = W.shape
    BS = B // NW
    peb = jnp.broadcast_to((_pos_encoding(L, D) / 8.0)[:, None, :], (L, BS, D))

    mesh = plsc.VectorSubcoreMesh(core_axis_name="c", subcore_axis_name="s")

    @functools.partial(
        pl.kernel,
        out_type=jax.ShapeDtypeStruct((L, D, B), jnp.float32),
        mesh=mesh,
        scratch_types=[
            pltpu.VMEM((BS, L), jnp.int32),
            pltpu.VMEM((L, BS), jnp.int32),
            pltpu.VMEM((NG, BS, D), jnp.float32),
            pltpu.VMEM((NT, D, BS), jnp.float32),
            pltpu.SemaphoreType.DMA((NG,)),
            pltpu.SemaphoreType.DMA((NG,)),
            pltpu.SemaphoreType.DMA((NT,)),
        ],
        compiler_params=pltpu.CompilerParams(use_tc_tiling_on_sc=False,
                                             needs_layout_passes=False),
    )
    def sc_run(w_hbm, x_hbm, peb_hbm, out_hbm,
               idxb, idxT, gbuf, tbuf, isem, gsem, osem):
        wid = lax.axis_index("s") * 2 + lax.axis_index("c")
        b0 = wid * BS
        rows = [jnp.arange(16, dtype=jnp.int32) + 16 * g for g in range(8)]

        pltpu.sync_copy(x_hbm.at[pl.ds(b0, BS)], idxb)

        def ib(l, carry):
            lane = jnp.full((16,), l, dtype=jnp.int32)
            for g in range(8):
                idxT[l, pl.ds(16 * g, 16)] = plsc.load_gather(
                    idxb, [rows[g], lane])
            return carry

        lax.fori_loop(0, L, ib, 0)

        def init_start(c, s):
            pltpu.async_copy(peb_hbm.at[c], gbuf.at[s], isem.at[s])

        def init_wait(c, s):
            pltpu.make_async_copy(peb_hbm.at[c], gbuf.at[s],
                                  isem.at[s]).wait()

        def gather_start(c, s):
            pltpu.async_copy(w_hbm.at[idxT.at[c]], gbuf.at[s], gsem.at[s],
                             add=True)

        def gather_wait(c, s):
            pltpu.make_async_copy(peb_hbm.at[0], gbuf.at[s],
                                  gsem.at[s]).wait()

        def out_start(c, t):
            pltpu.async_copy(tbuf.at[t], out_hbm.at[c, :, pl.ds(b0, BS)],
                             osem.at[t])

        def out_wait(c, t):
            pltpu.make_async_copy(tbuf.at[t], out_hbm.at[c, :, pl.ds(b0, BS)],
                                  osem.at[t]).wait()

        def transpose(s, t):
            def tp(k, carry):
                for dd in range(8):
                    d = k * 8 + dd
                    col = jnp.full((16,), d, dtype=jnp.int32)
                    for g in range(8):
                        v = plsc.load_gather(gbuf.at[s], [rows[g], col])
                        tbuf[t, d, pl.ds(16 * g, 16)] = v * 8.0
                return carry

            lax.fori_loop(0, D // 8, tp, 0)

        init_start(0, 0)
        init_start(1, 1)

        def body(o, carry):
            for i in range(NG):
                l = NG * o + i

                @pl.when(jnp.logical_and(l >= NG, l <= L + NG - 1))
                def _():
                    out_wait(l - NG, i % NT)

                @pl.when(jnp.logical_and(l >= 2, l <= L + 1))
                def _():
                    gather_wait(l - 2, (i + 2) % NG)
                    transpose((i + 2) % NG, i % NT)
                    out_start(l - 2, i % NT)

                @pl.when(l <= L - 1)
                def _():
                    init_wait(l, i)
                    gather_start(l, i)

                @pl.when(l <= L - 3)
                def _():
                    init_start(l + 2, (i + 2) % NG)

            return carry

        lax.fori_loop(0, (L + NG) // NG, body, 0)

    out = sc_run(W, x, peb)
    return jnp.transpose(out, (2, 0, 1))


# speedup vs baseline: 1.9534x; 1.9534x over previous
import functools

import jax
import jax.numpy as jnp
from jax import lax
from jax.experimental import pallas as pl
from jax.experimental.pallas import tpu as pltpu
from jax.experimental.pallas import tpu_sc as plsc

NW = 32
NG = 4
NT = 2


def _pos_encoding(length, d_model):
    depth = d_model / 2
    pos = jnp.arange(0, length, dtype=jnp.float32)[:, None]
    i = jnp.arange(0, depth, dtype=jnp.float32)
    angle = pos / jnp.power(10000.0, 2.0 * i / depth)
    return jnp.concatenate([jnp.sin(angle), jnp.cos(angle)], axis=-1)


def kernel(x, W):
    B, L = x.shape
    V, D = W.shape
    BS = B // NW
    peb = jnp.broadcast_to((_pos_encoding(L, D) / 8.0)[:, None, :], (L, BS, D))

    mesh = plsc.VectorSubcoreMesh(core_axis_name="c", subcore_axis_name="s")

    @functools.partial(
        pl.kernel,
        out_type=jax.ShapeDtypeStruct((L, D, B), jnp.float32),
        mesh=mesh,
        scratch_types=[
            pltpu.VMEM((BS, L), jnp.int32),
            pltpu.VMEM((L, BS), jnp.int32),
            pltpu.VMEM((NG, BS, D), jnp.float32),
            pltpu.VMEM((NT, D, BS), jnp.float32),
            pltpu.SemaphoreType.DMA((NG,)),
            pltpu.SemaphoreType.DMA((NG,)),
            pltpu.SemaphoreType.DMA((NT,)),
        ],
        compiler_params=pltpu.CompilerParams(use_tc_tiling_on_sc=False,
                                             needs_layout_passes=False),
    )
    def sc_run(w_hbm, x_hbm, peb_hbm, out_hbm,
               idxb, idxT, gbuf, tbuf, isem, gsem, osem):
        wid = lax.axis_index("s") * 2 + lax.axis_index("c")
        b0 = wid * BS
        rows = [jnp.arange(16, dtype=jnp.int32) + 16 * g for g in range(8)]

        pltpu.sync_copy(x_hbm.at[pl.ds(b0, BS)], idxb)

        def ib(l, carry):
            lane = jnp.full((16,), l, dtype=jnp.int32)
            for g in range(8):
                idxT[l, pl.ds(16 * g, 16)] = plsc.load_gather(
                    idxb, [rows[g], lane])
            return carry

        lax.fori_loop(0, L, ib, 0)

        def init_start(c, s):
            pltpu.async_copy(peb_hbm.at[c], gbuf.at[s], isem.at[s])

        def init_wait(c, s):
            pltpu.make_async_copy(peb_hbm.at[c], gbuf.at[s],
                                  isem.at[s]).wait()

        def gather_start(c, s):
            pltpu.async_copy(w_hbm.at[idxT.at[c]], gbuf.at[s], gsem.at[s],
                             add=True)

        def gather_wait(c, s):
            pltpu.make_async_copy(peb_hbm.at[0], gbuf.at[s],
                                  gsem.at[s]).wait()

        def out_start(c, t):
            pltpu.async_copy(tbuf.at[t], out_hbm.at[c, :, pl.ds(b0, BS)],
                             osem.at[t])

        def out_wait(c, t):
            pltpu.make_async_copy(tbuf.at[t], out_hbm.at[c, :, pl.ds(b0, BS)],
                                  osem.at[t]).wait()

        ar = jnp.arange(16, dtype=jnp.int32)
        diag = [lax.rem(ar + j, 16) for j in range(16)]

        def transpose(s, t):
            def tp(g, carry):
                brow = ar + 16 * g
                for h in range(D // 16):
                    for j in range(16):
                        dcol = diag[j] + 16 * h
                        v = plsc.load_gather(gbuf.at[s], [brow, dcol])
                        plsc.store_scatter(tbuf.at[t], [dcol, brow], v * 8.0)
                return carry

            lax.fori_loop(0, BS // 16, tp, 0)

        init_start(0, 0)
        init_start(1, 1)

        def body(o, carry):
            for i in range(NG):
                l = NG * o + i

                @pl.when(jnp.logical_and(l >= NG, l <= L + NG - 1))
                def _():
                    out_wait(l - NG, i % NT)

                @pl.when(jnp.logical_and(l >= 2, l <= L + 1))
                def _():
                    gather_wait(l - 2, (i + 2) % NG)
                    transpose((i + 2) % NG, i % NT)
                    out_start(l - 2, i % NT)

                @pl.when(l <= L - 1)
                def _():
                    init_wait(l, i)
                    gather_start(l, i)

                @pl.when(l <= L - 3)
                def _():
                    init_start(l + 2, (i + 2) % NG)

            return carry

        lax.fori_loop(0, (L + NG) // NG, body, 0)

    out = sc_run(W, x, peb)
    return jnp.transpose(out, (2, 0, 1))


# speedup vs baseline: 2.5529x; 1.3070x over previous
import functools

import jax
import jax.numpy as jnp
from jax import lax
from jax.experimental import pallas as pl
from jax.experimental.pallas import tpu as pltpu
from jax.experimental.pallas import tpu_sc as plsc

NW = 32
NG = 4
NT = 2


def _pos_encoding(length, d_model):
    depth = d_model / 2
    pos = jnp.arange(0, length, dtype=jnp.float32)[:, None]
    i = jnp.arange(0, depth, dtype=jnp.float32)
    angle = pos / jnp.power(10000.0, 2.0 * i / depth)
    return jnp.concatenate([jnp.sin(angle), jnp.cos(angle)], axis=-1)


def kernel(x, W):
    B, L = x.shape
    V, D = W.shape
    BS = B // NW
    peb = jnp.broadcast_to((_pos_encoding(L, D) / 8.0)[:, None, :], (L, BS, D))

    mesh = plsc.VectorSubcoreMesh(core_axis_name="c", subcore_axis_name="s")

    @functools.partial(
        pl.kernel,
        out_type=jax.ShapeDtypeStruct((L, D // 8, B // 128, 8, 128),
                                      jnp.float32),
        mesh=mesh,
        scratch_types=[
            pltpu.VMEM((BS, L), jnp.int32),
            pltpu.VMEM((L, BS), jnp.int32),
            pltpu.VMEM((NG, BS, D), jnp.float32),
            pltpu.VMEM((NT, D, BS), jnp.float32),
            pltpu.SemaphoreType.DMA((NG,)),
            pltpu.SemaphoreType.DMA((NG,)),
            pltpu.SemaphoreType.DMA((NT,)),
        ],
        compiler_params=pltpu.CompilerParams(use_tc_tiling_on_sc=False,
                                             needs_layout_passes=False),
    )
    def sc_run(w_hbm, x_hbm, peb_hbm, out_hbm,
               idxb, idxT, gbuf, tbuf, isem, gsem, osem):
        wid = lax.axis_index("s") * 2 + lax.axis_index("c")
        b0 = wid * BS
        rows = [jnp.arange(16, dtype=jnp.int32) + 16 * g for g in range(8)]

        pltpu.sync_copy(x_hbm.at[pl.ds(b0, BS)], idxb)

        def ib(l, carry):
            lane = jnp.full((16,), l, dtype=jnp.int32)
            for g in range(8):
                idxT[l, pl.ds(16 * g, 16)] = plsc.load_gather(
                    idxb, [rows[g], lane])
            return carry

        lax.fori_loop(0, L, ib, 0)

        def init_start(c, s):
            pltpu.async_copy(peb_hbm.at[c], gbuf.at[s], isem.at[s])

        def init_wait(c, s):
            pltpu.make_async_copy(peb_hbm.at[c], gbuf.at[s],
                                  isem.at[s]).wait()

        def gather_start(c, s):
            pltpu.async_copy(w_hbm.at[idxT.at[c]], gbuf.at[s], gsem.at[s],
                             add=True)

        def gather_wait(c, s):
            pltpu.make_async_copy(peb_hbm.at[0], gbuf.at[s],
                                  gsem.at[s]).wait()

        def out_start(c, t):
            for dh in range(D // 8):
                pltpu.async_copy(tbuf.at[t, pl.ds(8 * dh, 8)],
                                 out_hbm.at[c, dh, wid], osem.at[t])

        def out_wait(c, t):
            for dh in range(D // 8):
                pltpu.make_async_copy(tbuf.at[t, pl.ds(8 * dh, 8)],
                                      out_hbm.at[c, dh, wid],
                                      osem.at[t]).wait()

        ar = jnp.arange(16, dtype=jnp.int32)
        diag = [lax.rem(ar + j, 16) for j in range(16)]

        def transpose(s, t):
            def tp(g, carry):
                brow = ar + 16 * g
                for h in range(D // 16):
                    for j in range(16):
                        dcol = diag[j] + 16 * h
                        v = plsc.load_gather(gbuf.at[s], [brow, dcol])
                        plsc.store_scatter(tbuf.at[t], [dcol, brow], v * 8.0)
                return carry

            lax.fori_loop(0, BS // 16, tp, 0)

        init_start(0, 0)
        init_start(1, 1)

        def body(o, carry):
            for i in range(NG):
                l = NG * o + i

                @pl.when(jnp.logical_and(l >= NG, l <= L + NG - 1))
                def _():
                    out_wait(l - NG, i % NT)

                @pl.when(jnp.logical_and(l >= 2, l <= L + 1))
                def _():
                    gather_wait(l - 2, (i + 2) % NG)
                    transpose((i + 2) % NG, i % NT)
                    out_start(l - 2, i % NT)

                @pl.when(l <= L - 1)
                def _():
                    init_wait(l, i)
                    gather_start(l, i)

                @pl.when(l <= L - 3)
                def _():
                    init_start(l + 2, (i + 2) % NG)

            return carry

        lax.fori_loop(0, (L + NG) // NG, body, 0)

    out = sc_run(W, x, peb)
    return jnp.transpose(out, (2, 4, 0, 1, 3)).reshape(B, L, D)
